# Initial kernel scaffold; baseline (speedup 1.0000x reference)
import jax
import jax.numpy as jnp
from jax.experimental import pallas as pl


def kernel(x, Wq, Wk, Wv, Wo, bo, Wg, bg, w_k_compress, w_v_compress, w_pe_compress):
    raise NotImplementedError("write your pallas kernel here")


# speedup vs baseline: 12.4791x; 12.4791x over previous
import functools

import jax
import jax.numpy as jnp
import numpy as np
from jax.experimental import pallas as pl

S = 2048
D = 768
H = 12
HD = 64
CB = 16
STRIDE = 16
SB = 8
TOPK = 4
WIN = 256
NB = (S - CB) // STRIDE + 1
TQ = 256
NT = S // TQ
SCALE = 1.0 / np.sqrt(HD)
NEG = -1e30


def _softmax(x):
    m = jnp.max(x, axis=-1, keepdims=True)
    e = jnp.exp(x - m)
    return e / jnp.sum(e, axis=-1, keepdims=True)


def _dot(a, b):
    return jnp.dot(a.astype(jnp.bfloat16), b.astype(jnp.bfloat16),
                   preferred_element_type=jnp.float32)


def _dot_t(a, b):
    return jax.lax.dot_general(
        a.astype(jnp.bfloat16), b.astype(jnp.bfloat16),
        (((1,), (1,)), ((), ())), preferred_element_type=jnp.float32)


def _proj_kernel(x_ref, wq_ref, wk_ref, wv_ref, pk_ref, pv_ref,
                 wpe_ref, wg_ref, bg_ref,
                 q_out, k_out, v_out, ck_out, cv_out, g_out):
    xt = x_ref[...]
    for h in range(H):
        qh = _dot(xt, wq_ref[h])
        kh = _dot(xt, wk_ref[h])
        vh = _dot(xt, wv_ref[h])
        q_out[h] = qh
        k_out[h] = kh
        v_out[h] = vh
        kpe = kh + wpe_ref[h]
        vpe = vh + wpe_ref[h]
        ck_out[h] = _dot(pk_ref[...], kpe)
        cv_out[h] = _dot(pv_ref[...], vpe)
    gl = _dot(xt, wg_ref[...]) + bg_ref[0:1, :]
    g_out[...] = _softmax(gl)


def _attn_kernel(q_ref, k_ref, v_ref, ck_ref, cv_ref, g_ref, wo_ref, bo_ref,
                 o_ref):
    i = pl.program_id(0)
    q0 = i * TQ

    imp = jnp.zeros((TQ, NB), jnp.float32)
    comp = []
    for h in range(H):
        s = _dot_t(q_ref[h], ck_ref[h]) * SCALE
        imp = imp + s
        comp.append(_dot(_softmax(s), cv_ref[h]))

    lane = jax.lax.broadcasted_iota(jnp.int32, (TQ, NB), 1)
    hot = jnp.zeros((TQ, NB), jnp.float32)
    work = imp
    for _ in range(TOPK):
        mx = jnp.max(work, axis=-1, keepdims=True)
        pick = jnp.min(jnp.where(work == mx, lane, NB), axis=-1, keepdims=True)
        chosen = lane == pick
        hot = jnp.where(chosen, 1.0, hot)
        work = jnp.where(chosen, NEG, work)

    bb = jax.lax.broadcasted_iota(jnp.int32, (NB, S), 0)
    tt = jax.lax.broadcasted_iota(jnp.int32, (NB, S), 1)
    expand = jnp.where((tt // STRIDE == bb) & (tt % STRIDE < SB), 1.0, 0.0)
    selmask = jnp.dot(hot, expand, preferred_element_type=jnp.float32) > 0.5

    rows = q0 + jax.lax.broadcasted_iota(jnp.int32, (TQ, S), 0)
    cols = jax.lax.broadcasted_iota(jnp.int32, (TQ, S), 1)
    wmask = (cols <= rows) & (rows - cols <= WIN)

    g0 = g_ref[:, 0:1]
    g1 = g_ref[:, 1:2]
    g2 = g_ref[:, 2:3]
    acc = jnp.zeros((TQ, D), jnp.float32)
    for h in range(H):
        s = _dot_t(q_ref[h], k_ref[h]) * SCALE
        osel = _dot(_softmax(jnp.where(selmask, s, NEG)), v_ref[h])
        owin = _dot(_softmax(jnp.where(wmask, s, NEG)), v_ref[h])
        ch = g0 * comp[h] + g1 * osel + g2 * owin
        acc = acc + _dot(ch, wo_ref[h])
    o_ref[...] = acc + bo_ref[0:1, :]


@jax.jit
def _nsa_forward(x, Wq, Wk, Wv, Wo, bo, Wg, bg, wkc, wvc, wpe):
    f32 = jnp.float32
    x2 = x.reshape(S, D)
    wqh = Wq.reshape(D, H, HD).transpose(1, 0, 2)
    wkh = Wk.reshape(D, H, HD).transpose(1, 0, 2)
    wvh = Wv.reshape(D, H, HD).transpose(1, 0, 2)
    woh = Wo.reshape(H, HD, D)
    eye = jnp.eye(TQ // STRIDE, dtype=f32)
    pk = jnp.kron(eye, wkc.reshape(1, CB))
    pv = jnp.kron(eye, wvc.reshape(1, CB))
    wpe_h = wpe.reshape(CB, H, HD).transpose(1, 0, 2)
    wpe_t = jnp.tile(wpe_h, (1, TQ // CB, 1))
    wg_pad = jnp.zeros((D, 128), f32).at[:, :3].set(Wg)
    bg_pad = jnp.full((1, 128), -1e9, f32).at[0, :3].set(bg)
    bo_t = jnp.broadcast_to(bo.reshape(1, D), (8, D))

    full = lambda shape: pl.BlockSpec(shape, lambda i: (0,) * len(shape))
    qh, kh, vh, ckh, cvh, g = pl.pallas_call(
        _proj_kernel,
        grid=(NT,),
        in_specs=[
            pl.BlockSpec((TQ, D), lambda i: (i, 0)),
            full((H, D, HD)), full((H, D, HD)), full((H, D, HD)),
            full((CB, TQ)), full((CB, TQ)),
            full((H, TQ, HD)),
            full((D, 128)), full((1, 128)),
        ],
        out_specs=[
            pl.BlockSpec((H, TQ, HD), lambda i: (0, i, 0)),
            pl.BlockSpec((H, TQ, HD), lambda i: (0, i, 0)),
            pl.BlockSpec((H, TQ, HD), lambda i: (0, i, 0)),
            pl.BlockSpec((H, TQ // STRIDE, HD), lambda i: (0, i, 0)),
            pl.BlockSpec((H, TQ // STRIDE, HD), lambda i: (0, i, 0)),
            pl.BlockSpec((TQ, 128), lambda i: (i, 0)),
        ],
        out_shape=[
            jax.ShapeDtypeStruct((H, S, HD), f32),
            jax.ShapeDtypeStruct((H, S, HD), f32),
            jax.ShapeDtypeStruct((H, S, HD), f32),
            jax.ShapeDtypeStruct((H, NB, HD), f32),
            jax.ShapeDtypeStruct((H, NB, HD), f32),
            jax.ShapeDtypeStruct((S, 128), f32),
        ],
    )(x2, wqh, wkh, wvh, pk, pv, wpe_t, wg_pad, bg_pad)

    out = pl.pallas_call(
        _attn_kernel,
        grid=(NT,),
        in_specs=[
            pl.BlockSpec((H, TQ, HD), lambda i: (0, i, 0)),
            full((H, S, HD)), full((H, S, HD)),
            full((H, NB, HD)), full((H, NB, HD)),
            pl.BlockSpec((TQ, 128), lambda i: (i, 0)),
            full((H, HD, D)), full((8, D)),
        ],
        out_specs=pl.BlockSpec((TQ, D), lambda i: (i, 0)),
        out_shape=jax.ShapeDtypeStruct((S, D), f32),
    )(qh, kh, vh, ckh, cvh, g, woh, bo_t)
    return out.reshape(1, S, D)


def kernel(x, Wq, Wk, Wv, Wo, bo, Wg, bg, w_k_compress, w_v_compress,
           w_pe_compress):
    return _nsa_forward(x, Wq, Wk, Wv, Wo, bo, Wg, bg,
                        w_k_compress, w_v_compress, w_pe_compress)


# speedup vs baseline: 15.2437x; 1.2215x over previous
import jax
import jax.numpy as jnp
import numpy as np
from jax.experimental import pallas as pl

S = 2048
D = 768
H = 12
HD = 64
CB = 16
STRIDE = 16
SB = 8
TOPK = 4
WIN = 256
NB = (S - CB) // STRIDE + 1
TQ = 256
NT = S // TQ
SCALE = 1.0 / np.sqrt(HD)
NEG = -1e30


def _softmax(x):
    m = jnp.max(x, axis=-1, keepdims=True)
    e = jnp.exp(x - m)
    return e / jnp.sum(e, axis=-1, keepdims=True)


def _dot(a, b):
    return jnp.dot(a.astype(jnp.bfloat16), b.astype(jnp.bfloat16),
                   preferred_element_type=jnp.float32)


def _dot_t(a, b):
    return jax.lax.dot_general(
        a.astype(jnp.bfloat16), b.astype(jnp.bfloat16),
        (((1,), (1,)), ((), ())), preferred_element_type=jnp.float32)


def _proj_kernel(x_ref, wq_ref, wk_ref, wv_ref, pk_ref, pv_ref,
                 wpe_ref, wg_ref, bg_ref,
                 q_out, k_out, v_out, ck_out, cv_out, g_out):
    xt = x_ref[...]
    bf = jnp.bfloat16
    for h in range(H):
        qh = _dot(xt, wq_ref[h])
        kh = _dot(xt, wk_ref[h])
        vh = _dot(xt, wv_ref[h])
        q_out[h] = qh.astype(bf)
        k_out[h] = kh.astype(bf)
        v_out[h] = vh.astype(bf)
        ck_out[h] = _dot(pk_ref[...], kh + wpe_ref[h]).astype(bf)
        cv_out[h] = _dot(pv_ref[...], vh + wpe_ref[h]).astype(bf)
    gl = _dot(xt, wg_ref[...]) + bg_ref[0:1, :]
    g_out[...] = _softmax(gl)


def _attn_kernel(q_ref, k_ref, v_ref, ck_ref, cv_ref, g_ref, exp_ref,
                 wo_ref, bo_ref, o_ref):
    i = pl.program_id(0)
    q0 = i * TQ

    imp = jnp.zeros((TQ, NB), jnp.float32)
    comp = []
    for h in range(H):
        s = _dot_t(q_ref[h], ck_ref[h]) * SCALE
        imp = imp + s
        comp.append(_dot(_softmax(s), cv_ref[h]))

    lane = jax.lax.broadcasted_iota(jnp.int32, (TQ, NB), 1)
    hot = jnp.zeros((TQ, NB), jnp.float32)
    work = imp
    for _ in range(TOPK):
        mx = jnp.max(work, axis=-1, keepdims=True)
        pick = jnp.min(jnp.where(work == mx, lane, NB), axis=-1, keepdims=True)
        chosen = lane == pick
        hot = jnp.where(chosen, 1.0, hot)
        work = jnp.where(chosen, NEG, work)

    selmask = _dot(hot, exp_ref[...]) > 0.5

    rows = q0 + jax.lax.broadcasted_iota(jnp.int32, (TQ, S), 0)
    cols = jax.lax.broadcasted_iota(jnp.int32, (TQ, S), 1)
    wmask = (cols <= rows) & (rows - cols <= WIN)

    g0 = g_ref[:, 0:1]
    g1 = g_ref[:, 1:2]
    g2 = g_ref[:, 2:3]
    acc = jnp.zeros((TQ, D), jnp.float32)
    for h in range(H):
        s = _dot_t(q_ref[h], k_ref[h]) * SCALE
        m = jnp.max(s, axis=-1, keepdims=True)
        p = jnp.exp(s - m)
        psel = jnp.where(selmask, p, 0.0)
        pwin = jnp.where(wmask, p, 0.0)
        zs = jnp.sum(psel, axis=-1, keepdims=True)
        zw = jnp.sum(pwin, axis=-1, keepdims=True)
        osel = _dot(psel, v_ref[h]) / zs
        owin = _dot(pwin, v_ref[h]) / zw
        ch = g0 * comp[h] + g1 * osel + g2 * owin
        acc = acc + _dot(ch, wo_ref[h])
    o_ref[...] = acc + bo_ref[0:1, :]


@jax.jit
def _nsa_forward(x, Wq, Wk, Wv, Wo, bo, Wg, bg, wkc, wvc, wpe):
    f32 = jnp.float32
    bf = jnp.bfloat16
    x2 = x.reshape(S, D).astype(bf)
    wqh = Wq.reshape(D, H, HD).transpose(1, 0, 2).astype(bf)
    wkh = Wk.reshape(D, H, HD).transpose(1, 0, 2).astype(bf)
    wvh = Wv.reshape(D, H, HD).transpose(1, 0, 2).astype(bf)
    woh = Wo.reshape(H, HD, D).astype(bf)
    eye = jnp.eye(TQ // STRIDE, dtype=f32)
    pk = jnp.kron(eye, wkc.reshape(1, CB)).astype(bf)
    pv = jnp.kron(eye, wvc.reshape(1, CB)).astype(bf)
    wpe_h = wpe.reshape(CB, H, HD).transpose(1, 0, 2)
    wpe_t = jnp.tile(wpe_h, (1, TQ // CB, 1))
    wg_pad = jnp.zeros((D, 128), bf).at[:, :3].set(Wg.astype(bf))
    bg_pad = jnp.full((1, 128), -1e9, f32).at[0, :3].set(bg)
    bo_t = jnp.broadcast_to(bo.reshape(1, D), (8, D))
    bb = np.arange(NB)[:, None]
    tt = np.arange(S)[None, :]
    expand = jnp.asarray(((tt // STRIDE == bb) & (tt % STRIDE < SB)), dtype=bf)

    full = lambda shape: pl.BlockSpec(shape, lambda i: (0,) * len(shape))
    qh, kh, vh, ckh, cvh, g = pl.pallas_call(
        _proj_kernel,
        grid=(NT,),
        in_specs=[
            pl.BlockSpec((TQ, D), lambda i: (i, 0)),
            full((H, D, HD)), full((H, D, HD)), full((H, D, HD)),
            full((CB, TQ)), full((CB, TQ)),
            full((H, TQ, HD)),
            full((D, 128)), full((1, 128)),
        ],
        out_specs=[
            pl.BlockSpec((H, TQ, HD), lambda i: (0, i, 0)),
            pl.BlockSpec((H, TQ, HD), lambda i: (0, i, 0)),
            pl.BlockSpec((H, TQ, HD), lambda i: (0, i, 0)),
            pl.BlockSpec((H, TQ // STRIDE, HD), lambda i: (0, i, 0)),
            pl.BlockSpec((H, TQ // STRIDE, HD), lambda i: (0, i, 0)),
            pl.BlockSpec((TQ, 128), lambda i: (i, 0)),
        ],
        out_shape=[
            jax.ShapeDtypeStruct((H, S, HD), bf),
            jax.ShapeDtypeStruct((H, S, HD), bf),
            jax.ShapeDtypeStruct((H, S, HD), bf),
            jax.ShapeDtypeStruct((H, NB, HD), bf),
            jax.ShapeDtypeStruct((H, NB, HD), bf),
            jax.ShapeDtypeStruct((S, 128), f32),
        ],
    )(x2, wqh, wkh, wvh, pk, pv, wpe_t, wg_pad, bg_pad)

    out = pl.pallas_call(
        _attn_kernel,
        grid=(NT,),
        in_specs=[
            pl.BlockSpec((H, TQ, HD), lambda i: (0, i, 0)),
            full((H, S, HD)), full((H, S, HD)),
            full((H, NB, HD)), full((H, NB, HD)),
            pl.BlockSpec((TQ, 128), lambda i: (i, 0)),
            full((NB, S)),
            full((H, HD, D)), full((8, D)),
        ],
        out_specs=pl.BlockSpec((TQ, D), lambda i: (i, 0)),
        out_shape=jax.ShapeDtypeStruct((S, D), f32),
    )(qh, kh, vh, ckh, cvh, g, expand, woh, bo_t)
    return out.reshape(1, S, D)


def kernel(x, Wq, Wk, Wv, Wo, bo, Wg, bg, w_k_compress, w_v_compress,
           w_pe_compress):
    return _nsa_forward(x, Wq, Wk, Wv, Wo, bo, Wg, bg,
                        w_k_compress, w_v_compress, w_pe_compress)


# speedup vs baseline: 18.7384x; 1.2293x over previous
import jax
import jax.numpy as jnp
import numpy as np
from jax.experimental import pallas as pl

S = 2048
D = 768
H = 12
HD = 64
CB = 16
STRIDE = 16
SB = 8
TOPK = 4
WIN = 256
NB = (S - CB) // STRIDE + 1
TQ = 256
NT = S // TQ
SCALE = 1.0 / np.sqrt(HD)
NEG = -1e30


def _softmax(x):
    m = jnp.max(x, axis=-1, keepdims=True)
    e = jnp.exp(x - m)
    return e / jnp.sum(e, axis=-1, keepdims=True)


def _dot(a, b):
    return jnp.dot(a.astype(jnp.bfloat16), b.astype(jnp.bfloat16),
                   preferred_element_type=jnp.float32)


def _dot_t(a, b):
    return jax.lax.dot_general(
        a.astype(jnp.bfloat16), b.astype(jnp.bfloat16),
        (((1,), (1,)), ((), ())), preferred_element_type=jnp.float32)


def _proj_kernel(x_ref, wq_ref, wk_ref, wv_ref, pk_ref, pv_ref,
                 wpe_ref, wg_ref, bg_ref,
                 q_out, k_out, v_out, ck_out, cv_out, g_out):
    xt = x_ref[...]
    bf = jnp.bfloat16
    one_s = (jax.lax.broadcasted_iota(jnp.int32, (TQ, HD), 1) == 0).astype(bf)
    one_c = (jax.lax.broadcasted_iota(
        jnp.int32, (TQ // STRIDE, HD), 1) == 0).astype(bf)
    for h in range(H):
        qh = _dot(xt, wq_ref[h])
        kh = _dot(xt, wk_ref[h])
        vh = _dot(xt, wv_ref[h])
        q_out[h] = qh.astype(bf)
        k_out[h] = kh.astype(bf)
        v_out[h] = jnp.concatenate([vh.astype(bf), one_s], axis=-1)
        ck_out[h] = _dot(pk_ref[...], kh + wpe_ref[h]).astype(bf)
        cv_out[h] = jnp.concatenate(
            [_dot(pv_ref[...], vh + wpe_ref[h]).astype(bf), one_c], axis=-1)
    gl = _dot(xt, wg_ref[...]) + bg_ref[0:1, :]
    g_out[...] = _softmax(gl)


def _attn_kernel(q_ref, k_ref, v_ref, ck_ref, cv_ref, g_ref, exp_ref,
                 wo_ref, bo_ref, o_ref):
    bf = jnp.bfloat16
    f32 = jnp.float32
    q0 = pl.program_id(0) * TQ

    imp = jnp.zeros((TQ, NB), jnp.float32)
    comp = []
    for h in range(H):
        s = _dot_t(q_ref[h], ck_ref[h]) * SCALE
        imp = imp + s
        pc = jnp.exp(s - jnp.max(s, axis=-1, keepdims=True)).astype(bf)
        rc = jnp.dot(pc, cv_ref[h], preferred_element_type=f32)
        comp.append(rc[:, :HD] / rc[:, HD:HD + 1])

    lane = jax.lax.broadcasted_iota(jnp.int32, (TQ, NB), 1)
    hot = jnp.zeros((TQ, NB), jnp.float32)
    work = imp
    for _ in range(TOPK):
        mx = jnp.max(work, axis=-1, keepdims=True)
        pick = jnp.min(jnp.where(work == mx, lane, NB), axis=-1, keepdims=True)
        chosen = lane == pick
        hot = jnp.where(chosen, 1.0, hot)
        work = jnp.where(chosen, NEG, work)

    selmask = _dot(hot, exp_ref[...]) > 0.5

    rows = q0 + jax.lax.broadcasted_iota(jnp.int32, (TQ, S), 0)
    cols = jax.lax.broadcasted_iota(jnp.int32, (TQ, S), 1)
    wmask = (cols <= rows) & (rows - cols <= WIN)

    g0 = g_ref[:, 0:1]
    g1 = g_ref[:, 1:2]
    g2 = g_ref[:, 2:3]
    chs = []
    for h in range(H):
        s = _dot_t(q_ref[h], k_ref[h]) * SCALE
        m = jnp.max(s, axis=-1, keepdims=True)
        p = jnp.exp(s - m).astype(bf)
        zero = jnp.zeros((), bf)
        psel = jnp.where(selmask, p, zero)
        pwin = jnp.where(wmask, p, zero)
        rsel = jnp.dot(psel, v_ref[h], preferred_element_type=f32)
        rwin = jnp.dot(pwin, v_ref[h], preferred_element_type=f32)
        osel = rsel[:, :HD] / rsel[:, HD:HD + 1]
        owin = rwin[:, :HD] / rwin[:, HD:HD + 1]
        chs.append((g0 * comp[h] + g1 * osel + g2 * owin).astype(bf))
    cat = jnp.concatenate(chs, axis=-1)
    o_ref[...] = jnp.dot(cat, wo_ref[...],
                         preferred_element_type=f32) + bo_ref[0:1, :]


@jax.jit
def _nsa_forward(x, Wq, Wk, Wv, Wo, bo, Wg, bg, wkc, wvc, wpe):
    f32 = jnp.float32
    bf = jnp.bfloat16
    x2 = x.reshape(S, D).astype(bf)
    wqh = Wq.reshape(D, H, HD).transpose(1, 0, 2).astype(bf)
    wkh = Wk.reshape(D, H, HD).transpose(1, 0, 2).astype(bf)
    wvh = Wv.reshape(D, H, HD).transpose(1, 0, 2).astype(bf)
    wo2 = Wo.reshape(D, D).astype(bf)
    eye = jnp.eye(TQ // STRIDE, dtype=f32)
    pk = jnp.kron(eye, wkc.reshape(1, CB)).astype(bf)
    pv = jnp.kron(eye, wvc.reshape(1, CB)).astype(bf)
    wpe_h = wpe.reshape(CB, H, HD).transpose(1, 0, 2)
    wpe_t = jnp.tile(wpe_h, (1, TQ // CB, 1))
    wg_pad = jnp.zeros((D, 128), bf).at[:, :3].set(Wg.astype(bf))
    bg_pad = jnp.full((1, 128), -1e9, f32).at[0, :3].set(bg)
    bo_t = jnp.broadcast_to(bo.reshape(1, D), (8, D))
    bb = np.arange(NB)[:, None]
    tt = np.arange(S)[None, :]
    expand = jnp.asarray(((tt // STRIDE == bb) & (tt % STRIDE < SB)), dtype=bf)

    full = lambda shape: pl.BlockSpec(shape, lambda i: (0,) * len(shape))
    qh, kh, vh, ckh, cvh, g = pl.pallas_call(
        _proj_kernel,
        grid=(NT,),
        in_specs=[
            pl.BlockSpec((TQ, D), lambda i: (i, 0)),
            full((H, D, HD)), full((H, D, HD)), full((H, D, HD)),
            full((CB, TQ)), full((CB, TQ)),
            full((H, TQ, HD)),
            full((D, 128)), full((1, 128)),
        ],
        out_specs=[
            pl.BlockSpec((H, TQ, HD), lambda i: (0, i, 0)),
            pl.BlockSpec((H, TQ, HD), lambda i: (0, i, 0)),
            pl.BlockSpec((H, TQ, 2 * HD), lambda i: (0, i, 0)),
            pl.BlockSpec((H, TQ // STRIDE, HD), lambda i: (0, i, 0)),
            pl.BlockSpec((H, TQ // STRIDE, 2 * HD), lambda i: (0, i, 0)),
            pl.BlockSpec((TQ, 128), lambda i: (i, 0)),
        ],
        out_shape=[
            jax.ShapeDtypeStruct((H, S, HD), bf),
            jax.ShapeDtypeStruct((H, S, HD), bf),
            jax.ShapeDtypeStruct((H, S, 2 * HD), bf),
            jax.ShapeDtypeStruct((H, NB, HD), bf),
            jax.ShapeDtypeStruct((H, NB, 2 * HD), bf),
            jax.ShapeDtypeStruct((S, 128), f32),
        ],
    )(x2, wqh, wkh, wvh, pk, pv, wpe_t, wg_pad, bg_pad)

    out = pl.pallas_call(
        _attn_kernel,
        grid=(NT,),
        in_specs=[
            pl.BlockSpec((H, TQ, HD), lambda i: (0, i, 0)),
            full((H, S, HD)), full((H, S, 2 * HD)),
            full((H, NB, HD)), full((H, NB, 2 * HD)),
            pl.BlockSpec((TQ, 128), lambda i: (i, 0)),
            full((NB, S)),
            full((D, D)), full((8, D)),
        ],
        out_specs=pl.BlockSpec((TQ, D), lambda i: (i, 0)),
        out_shape=jax.ShapeDtypeStruct((S, D), f32),
    )(qh, kh, vh, ckh, cvh, g, expand, wo2, bo_t)
    return out.reshape(1, S, D)


def kernel(x, Wq, Wk, Wv, Wo, bo, Wg, bg, w_k_compress, w_v_compress,
           w_pe_compress):
    return _nsa_forward(x, Wq, Wk, Wv, Wo, bo, Wg, bg,
                        w_k_compress, w_v_compress, w_pe_compress)


# speedup vs baseline: 20.1047x; 1.0729x over previous
import jax
import jax.numpy as jnp
import numpy as np
from jax.experimental import pallas as pl

S = 2048
D = 768
H = 12
HD = 64
CB = 16
STRIDE = 16
SB = 8
TOPK = 4
WIN = 256
NB = (S - CB) // STRIDE + 1
TQ = 256
NT = S // TQ
SCALE = 1.0 / np.sqrt(HD)
NEG = -1e30
BAND = WIN + TQ


def _softmax(x):
    m = jnp.max(x, axis=-1, keepdims=True)
    e = jnp.exp(x - m)
    return e / jnp.sum(e, axis=-1, keepdims=True)


def _dot(a, b):
    return jnp.dot(a.astype(jnp.bfloat16), b.astype(jnp.bfloat16),
                   preferred_element_type=jnp.float32)


def _dot_t(a, b):
    return jax.lax.dot_general(
        a.astype(jnp.bfloat16), b.astype(jnp.bfloat16),
        (((1,), (1,)), ((), ())), preferred_element_type=jnp.float32)


def _proj_kernel(x_ref, wq_ref, wk_ref, wv_ref, pk_ref, pv_ref,
                 wpe_ref, wg_ref, bg_ref,
                 q_out, k_out, v_out, ck_out, cv_out, g_out):
    xt = x_ref[...]
    bf = jnp.bfloat16
    one_s = (jax.lax.broadcasted_iota(jnp.int32, (TQ, HD), 1) == 0).astype(bf)
    one_c = (jax.lax.broadcasted_iota(
        jnp.int32, (TQ // STRIDE, HD), 1) == 0).astype(bf)
    for h in range(H):
        qh = _dot(xt, wq_ref[h])
        kh = _dot(xt, wk_ref[h])
        vh = _dot(xt, wv_ref[h])
        q_out[h] = qh.astype(bf)
        k_out[h] = kh.astype(bf)
        v_out[h] = jnp.concatenate([vh.astype(bf), one_s], axis=-1)
        ck_out[h] = _dot(pk_ref[...], kh + wpe_ref[h]).astype(bf)
        cv_out[h] = jnp.concatenate(
            [_dot(pv_ref[...], vh + wpe_ref[h]).astype(bf), one_c], axis=-1)
    gl = _dot(xt, wg_ref[...]) + bg_ref[0:1, :]
    g_out[...] = _softmax(gl)


def _attn_kernel(q_ref, k_ref, v_ref, ck_ref, cv_ref, g_ref, exp_ref,
                 wo_ref, bo_ref, o_ref):
    bf = jnp.bfloat16
    f32 = jnp.float32
    q0 = pl.program_id(0) * TQ

    imp = jnp.zeros((TQ, NB), jnp.float32)
    comp = []
    for h in range(H):
        s = _dot_t(q_ref[h], ck_ref[h]) * SCALE
        imp = imp + s
        pc = jnp.exp(s - jnp.max(s, axis=-1, keepdims=True)).astype(bf)
        rc = jnp.dot(pc, cv_ref[h], preferred_element_type=f32)
        comp.append(rc[:, :HD] / rc[:, HD:HD + 1])

    lane = jax.lax.broadcasted_iota(jnp.int32, (TQ, NB), 1)
    hot = jnp.zeros((TQ, NB), jnp.float32)
    work = imp
    for _ in range(TOPK):
        mx = jnp.max(work, axis=-1, keepdims=True)
        pick = jnp.min(jnp.where(work == mx, lane, NB), axis=-1, keepdims=True)
        chosen = lane == pick
        hot = jnp.where(chosen, 1.0, hot)
        work = jnp.where(chosen, NEG, work)

    selmask = _dot(hot, exp_ref[...]) > 0.5

    start = pl.multiple_of(jnp.maximum(q0 - WIN, 0), TQ)
    brows = q0 + jax.lax.broadcasted_iota(jnp.int32, (TQ, BAND), 0)
    bcols = start + jax.lax.broadcasted_iota(jnp.int32, (TQ, BAND), 1)
    bwmask = (bcols <= brows) & (brows - bcols <= WIN)

    g0 = g_ref[:, 0:1]
    g1 = g_ref[:, 1:2]
    g2 = g_ref[:, 2:3]
    chs = []
    for h in range(H):
        s = _dot_t(q_ref[h], k_ref[h]) * SCALE
        m = jnp.max(s, axis=-1, keepdims=True)
        p = jnp.exp(s - m).astype(bf)
        zero = jnp.zeros((), bf)
        psel = jnp.where(selmask, p, zero)
        kband = k_ref[h, pl.ds(start, BAND), :]
        vband = v_ref[h, pl.ds(start, BAND), :]
        sband = _dot_t(q_ref[h], kband) * SCALE
        pwin = jnp.where(bwmask, jnp.exp(sband - m).astype(bf), zero)
        rsel = jnp.dot(psel, v_ref[h], preferred_element_type=f32)
        rwin = jnp.dot(pwin, vband, preferred_element_type=f32)
        osel = rsel[:, :HD] / rsel[:, HD:HD + 1]
        owin = rwin[:, :HD] / rwin[:, HD:HD + 1]
        chs.append((g0 * comp[h] + g1 * osel + g2 * owin).astype(bf))
    cat = jnp.concatenate(chs, axis=-1)
    o_ref[...] = jnp.dot(cat, wo_ref[...],
                         preferred_element_type=f32) + bo_ref[0:1, :]


@jax.jit
def _nsa_forward(x, Wq, Wk, Wv, Wo, bo, Wg, bg, wkc, wvc, wpe):
    f32 = jnp.float32
    bf = jnp.bfloat16
    x2 = x.reshape(S, D).astype(bf)
    wqh = Wq.reshape(D, H, HD).transpose(1, 0, 2).astype(bf)
    wkh = Wk.reshape(D, H, HD).transpose(1, 0, 2).astype(bf)
    wvh = Wv.reshape(D, H, HD).transpose(1, 0, 2).astype(bf)
    wo2 = Wo.reshape(D, D).astype(bf)
    eye = jnp.eye(TQ // STRIDE, dtype=f32)
    pk = jnp.kron(eye, wkc.reshape(1, CB)).astype(bf)
    pv = jnp.kron(eye, wvc.reshape(1, CB)).astype(bf)
    wpe_h = wpe.reshape(CB, H, HD).transpose(1, 0, 2)
    wpe_t = jnp.tile(wpe_h, (1, TQ // CB, 1))
    wg_pad = jnp.zeros((D, 128), bf).at[:, :3].set(Wg.astype(bf))
    bg_pad = jnp.full((1, 128), -1e9, f32).at[0, :3].set(bg)
    bo_t = jnp.broadcast_to(bo.reshape(1, D), (8, D))
    bb = np.arange(NB)[:, None]
    tt = np.arange(S)[None, :]
    expand = jnp.asarray(((tt // STRIDE == bb) & (tt % STRIDE < SB)), dtype=bf)

    full = lambda shape: pl.BlockSpec(shape, lambda i: (0,) * len(shape))
    qh, kh, vh, ckh, cvh, g = pl.pallas_call(
        _proj_kernel,
        grid=(NT,),
        in_specs=[
            pl.BlockSpec((TQ, D), lambda i: (i, 0)),
            full((H, D, HD)), full((H, D, HD)), full((H, D, HD)),
            full((CB, TQ)), full((CB, TQ)),
            full((H, TQ, HD)),
            full((D, 128)), full((1, 128)),
        ],
        out_specs=[
            pl.BlockSpec((H, TQ, HD), lambda i: (0, i, 0)),
            pl.BlockSpec((H, TQ, HD), lambda i: (0, i, 0)),
            pl.BlockSpec((H, TQ, 2 * HD), lambda i: (0, i, 0)),
            pl.BlockSpec((H, TQ // STRIDE, HD), lambda i: (0, i, 0)),
            pl.BlockSpec((H, TQ // STRIDE, 2 * HD), lambda i: (0, i, 0)),
            pl.BlockSpec((TQ, 128), lambda i: (i, 0)),
        ],
        out_shape=[
            jax.ShapeDtypeStruct((H, S, HD), bf),
            jax.ShapeDtypeStruct((H, S, HD), bf),
            jax.ShapeDtypeStruct((H, S, 2 * HD), bf),
            jax.ShapeDtypeStruct((H, NB, HD), bf),
            jax.ShapeDtypeStruct((H, NB, 2 * HD), bf),
            jax.ShapeDtypeStruct((S, 128), f32),
        ],
    )(x2, wqh, wkh, wvh, pk, pv, wpe_t, wg_pad, bg_pad)

    out = pl.pallas_call(
        _attn_kernel,
        grid=(NT,),
        in_specs=[
            pl.BlockSpec((H, TQ, HD), lambda i: (0, i, 0)),
            full((H, S, HD)), full((H, S, 2 * HD)),
            full((H, NB, HD)), full((H, NB, 2 * HD)),
            pl.BlockSpec((TQ, 128), lambda i: (i, 0)),
            full((NB, S)),
            full((D, D)), full((8, D)),
        ],
        out_specs=pl.BlockSpec((TQ, D), lambda i: (i, 0)),
        out_shape=jax.ShapeDtypeStruct((S, D), f32),
    )(qh, kh, vh, ckh, cvh, g, expand, wo2, bo_t)
    return out.reshape(1, S, D)


def kernel(x, Wq, Wk, Wv, Wo, bo, Wg, bg, w_k_compress, w_v_compress,
           w_pe_compress):
    return _nsa_forward(x, Wq, Wk, Wv, Wo, bo, Wg, bg,
                        w_k_compress, w_v_compress, w_pe_compress)


# speedup vs baseline: 20.8933x; 1.0392x over previous
import jax
import jax.numpy as jnp
import numpy as np
from jax.experimental import pallas as pl

S = 2048
D = 768
H = 12
HD = 64
CB = 16
STRIDE = 16
SB = 8
TOPK = 4
WIN = 256
NB = (S - CB) // STRIDE + 1
TQ = 256
NT = S // TQ
SCALE = 1.0 / np.sqrt(HD)
NEG = -1e30
BAND = WIN + TQ


def _softmax(x):
    m = jnp.max(x, axis=-1, keepdims=True)
    e = jnp.exp(x - m)
    return e / jnp.sum(e, axis=-1, keepdims=True)


def _dot(a, b):
    return jnp.dot(a.astype(jnp.bfloat16), b.astype(jnp.bfloat16),
                   preferred_element_type=jnp.float32)


def _dot_t(a, b):
    return jax.lax.dot_general(
        a.astype(jnp.bfloat16), b.astype(jnp.bfloat16),
        (((1,), (1,)), ((), ())), preferred_element_type=jnp.float32)


def _proj_kernel(x_ref, wq_ref, wk_ref, wv_ref, pk_ref, pv_ref,
                 wpe_ref, wg_ref, bg_ref,
                 q_out, k_out, v_out, ck_out, cv_out, g_out):
    xt = x_ref[...]
    bf = jnp.bfloat16
    one_s = (jax.lax.broadcasted_iota(jnp.int32, (TQ, HD), 1) == 0).astype(bf)
    one_c = (jax.lax.broadcasted_iota(
        jnp.int32, (TQ // STRIDE, HD), 1) == 0).astype(bf)
    for h in range(H):
        qh = _dot(xt, wq_ref[h])
        kh = _dot(xt, wk_ref[h])
        vh = _dot(xt, wv_ref[h])
        q_out[h] = (qh * SCALE).astype(bf)
        k_out[h] = kh.astype(bf)
        v_out[h] = jnp.concatenate([vh.astype(bf), one_s], axis=-1)
        ck_out[h] = _dot(pk_ref[...], kh + wpe_ref[h]).astype(bf)
        cv_out[h] = jnp.concatenate(
            [_dot(pv_ref[...], vh + wpe_ref[h]).astype(bf), one_c], axis=-1)
    gl = _dot(xt, wg_ref[...]) + bg_ref[0:1, :]
    g_out[...] = _softmax(gl)


def _attn_kernel(q_ref, k_ref, v_ref, ck_ref, cv_ref, g_ref, exp_ref,
                 wo_ref, bo_ref, o_ref):
    bf = jnp.bfloat16
    f32 = jnp.float32
    q0 = pl.program_id(0) * TQ

    imp = jnp.zeros((TQ, NB), jnp.float32)
    comp = []
    for h in range(H):
        s = _dot_t(q_ref[h], ck_ref[h])
        imp = imp + s
        pc = jnp.exp(s - jnp.max(s, axis=-1, keepdims=True)).astype(bf)
        rc = jnp.dot(pc, cv_ref[h], preferred_element_type=f32)
        comp.append(rc[:, :HD] / rc[:, HD:HD + 1])

    lane = jax.lax.broadcasted_iota(jnp.int32, (TQ, NB), 1)
    hot = jnp.zeros((TQ, NB), jnp.float32)
    work = imp
    for _ in range(TOPK):
        mx = jnp.max(work, axis=-1, keepdims=True)
        pick = jnp.min(jnp.where(work == mx, lane, NB), axis=-1, keepdims=True)
        chosen = lane == pick
        hot = jnp.where(chosen, 1.0, hot)
        work = jnp.where(chosen, NEG, work)

    selmask = _dot(hot, exp_ref[...]) > 0.5

    start = pl.multiple_of(jnp.maximum(q0 - WIN, 0), TQ)
    brows = q0 + jax.lax.broadcasted_iota(jnp.int32, (TQ, BAND), 0)
    bcols = start + jax.lax.broadcasted_iota(jnp.int32, (TQ, BAND), 1)
    bwmask = (bcols <= brows) & (brows - bcols <= WIN)

    g0 = g_ref[:, 0:1]
    g1 = g_ref[:, 1:2]
    g2 = g_ref[:, 2:3]
    chs = []
    for h in range(H):
        s = _dot_t(q_ref[h], k_ref[h])
        m = jnp.max(s, axis=-1, keepdims=True)
        p = jnp.exp(s - m).astype(bf)
        zero = jnp.zeros((), bf)
        psel = jnp.where(selmask, p, zero)
        kband = k_ref[h, pl.ds(start, BAND), :]
        vband = v_ref[h, pl.ds(start, BAND), :]
        sband = _dot_t(q_ref[h], kband)
        pwin = jnp.where(bwmask, jnp.exp(sband - m).astype(bf), zero)
        rsel = jnp.dot(psel, v_ref[h], preferred_element_type=f32)
        rwin = jnp.dot(pwin, vband, preferred_element_type=f32)
        osel = rsel[:, :HD] / rsel[:, HD:HD + 1]
        owin = rwin[:, :HD] / rwin[:, HD:HD + 1]
        chs.append((g0 * comp[h] + g1 * osel + g2 * owin).astype(bf))
    cat = jnp.concatenate(chs, axis=-1)
    o_ref[...] = jnp.dot(cat, wo_ref[...],
                         preferred_element_type=f32) + bo_ref[0:1, :]


@jax.jit
def _nsa_forward(x, Wq, Wk, Wv, Wo, bo, Wg, bg, wkc, wvc, wpe):
    f32 = jnp.float32
    bf = jnp.bfloat16
    x2 = x.reshape(S, D).astype(bf)
    wqh = Wq.reshape(D, H, HD).transpose(1, 0, 2).astype(bf)
    wkh = Wk.reshape(D, H, HD).transpose(1, 0, 2).astype(bf)
    wvh = Wv.reshape(D, H, HD).transpose(1, 0, 2).astype(bf)
    wo2 = Wo.reshape(D, D).astype(bf)
    eye = jnp.eye(TQ // STRIDE, dtype=f32)
    pk = jnp.kron(eye, wkc.reshape(1, CB)).astype(bf)
    pv = jnp.kron(eye, wvc.reshape(1, CB)).astype(bf)
    wpe_h = wpe.reshape(CB, H, HD).transpose(1, 0, 2)
    wpe_t = jnp.tile(wpe_h, (1, TQ // CB, 1))
    wg_pad = jnp.zeros((D, 128), bf).at[:, :3].set(Wg.astype(bf))
    bg_pad = jnp.full((1, 128), -1e9, f32).at[0, :3].set(bg)
    bo_t = jnp.broadcast_to(bo.reshape(1, D), (8, D))
    bb = np.arange(NB)[:, None]
    tt = np.arange(S)[None, :]
    expand = jnp.asarray(((tt // STRIDE == bb) & (tt % STRIDE < SB)), dtype=bf)

    full = lambda shape: pl.BlockSpec(shape, lambda i: (0,) * len(shape))
    qh, kh, vh, ckh, cvh, g = pl.pallas_call(
        _proj_kernel,
        grid=(NT,),
        in_specs=[
            pl.BlockSpec((TQ, D), lambda i: (i, 0)),
            full((H, D, HD)), full((H, D, HD)), full((H, D, HD)),
            full((CB, TQ)), full((CB, TQ)),
            full((H, TQ, HD)),
            full((D, 128)), full((1, 128)),
        ],
        out_specs=[
            pl.BlockSpec((H, TQ, HD), lambda i: (0, i, 0)),
            pl.BlockSpec((H, TQ, HD), lambda i: (0, i, 0)),
            pl.BlockSpec((H, TQ, 2 * HD), lambda i: (0, i, 0)),
            pl.BlockSpec((H, TQ // STRIDE, HD), lambda i: (0, i, 0)),
            pl.BlockSpec((H, TQ // STRIDE, 2 * HD), lambda i: (0, i, 0)),
            pl.BlockSpec((TQ, 128), lambda i: (i, 0)),
        ],
        out_shape=[
            jax.ShapeDtypeStruct((H, S, HD), bf),
            jax.ShapeDtypeStruct((H, S, HD), bf),
            jax.ShapeDtypeStruct((H, S, 2 * HD), bf),
            jax.ShapeDtypeStruct((H, NB, HD), bf),
            jax.ShapeDtypeStruct((H, NB, 2 * HD), bf),
            jax.ShapeDtypeStruct((S, 128), f32),
        ],
    )(x2, wqh, wkh, wvh, pk, pv, wpe_t, wg_pad, bg_pad)

    out = pl.pallas_call(
        _attn_kernel,
        grid=(NT,),
        in_specs=[
            pl.BlockSpec((H, TQ, HD), lambda i: (0, i, 0)),
            full((H, S, HD)), full((H, S, 2 * HD)),
            full((H, NB, HD)), full((H, NB, 2 * HD)),
            pl.BlockSpec((TQ, 128), lambda i: (i, 0)),
            full((NB, S)),
            full((D, D)), full((8, D)),
        ],
        out_specs=pl.BlockSpec((TQ, D), lambda i: (i, 0)),
        out_shape=jax.ShapeDtypeStruct((S, D), f32),
    )(qh, kh, vh, ckh, cvh, g, expand, wo2, bo_t)
    return out.reshape(1, S, D)


def kernel(x, Wq, Wk, Wv, Wo, bo, Wg, bg, w_k_compress, w_v_compress,
           w_pe_compress):
    return _nsa_forward(x, Wq, Wk, Wv, Wo, bo, Wg, bg,
                        w_k_compress, w_v_compress, w_pe_compress)


# speedup vs baseline: 27.3301x; 1.3081x over previous
import jax
import jax.numpy as jnp
import numpy as np
from jax.experimental import pallas as pl

S = 2048
D = 768
H = 12
HD = 64
CB = 16
STRIDE = 16
SB = 8
TOPK = 4
WIN = 256
NB = (S - CB) // STRIDE + 1
TQ = 256
NT = S // TQ
SCALE = 1.0 / np.sqrt(HD)
NEG = -1e30
BAND = WIN + TQ


def _softmax(x):
    m = jnp.max(x, axis=-1, keepdims=True)
    e = jnp.exp(x - m)
    return e / jnp.sum(e, axis=-1, keepdims=True)


def _dot(a, b):
    return jnp.dot(a.astype(jnp.bfloat16), b.astype(jnp.bfloat16),
                   preferred_element_type=jnp.float32)


def _dot_t(a, b):
    return jax.lax.dot_general(
        a.astype(jnp.bfloat16), b.astype(jnp.bfloat16),
        (((1,), (1,)), ((), ())), preferred_element_type=jnp.float32)


def _proj_kernel(x_ref, wq_ref, wk_ref, wv_ref, pk_ref, pv_ref,
                 wpe_ref, wg_ref, bg_ref,
                 q_out, k_out, v_out, ck_out, cv_out, g_out):
    xt = x_ref[...]
    bf = jnp.bfloat16
    one_s = (jax.lax.broadcasted_iota(jnp.int32, (TQ, HD), 1) == 0).astype(bf)
    one_c = (jax.lax.broadcasted_iota(
        jnp.int32, (TQ // STRIDE, HD), 1) == 0).astype(bf)
    qa = _dot(xt, wq_ref[...])
    ka = _dot(xt, wk_ref[...])
    va = _dot(xt, wv_ref[...])
    for h in range(H):
        qh = qa[:, h * HD:(h + 1) * HD]
        kh = ka[:, h * HD:(h + 1) * HD]
        vh = va[:, h * HD:(h + 1) * HD]
        q_out[h] = (qh * SCALE).astype(bf)
        k_out[h] = kh.astype(bf)
        v_out[h] = jnp.concatenate([vh.astype(bf), one_s], axis=-1)
        ck_out[h] = _dot(pk_ref[...], kh + wpe_ref[h]).astype(bf)
        cv_out[h] = jnp.concatenate(
            [_dot(pv_ref[...], vh + wpe_ref[h]).astype(bf), one_c], axis=-1)
    gl = _dot(xt, wg_ref[...]) + bg_ref[0:1, :]
    g_out[...] = _softmax(gl)


def _attn_kernel(q_ref, k_ref, v_ref, ck_ref, cv_ref, g_ref, exp_ref,
                 wo_ref, bo_ref, o_ref):
    bf = jnp.bfloat16
    f32 = jnp.float32
    q0 = pl.program_id(0) * TQ

    imp = jnp.zeros((TQ, NB), jnp.float32)
    comp = []
    for h in range(H):
        s = _dot_t(q_ref[h], ck_ref[h])
        imp = imp + s
        pc = jnp.exp(s - jnp.max(s, axis=-1, keepdims=True)).astype(bf)
        rc = jnp.dot(pc, cv_ref[h], preferred_element_type=f32)
        comp.append(rc[:, :HD] / rc[:, HD:HD + 1])

    lane = jax.lax.broadcasted_iota(jnp.int32, (TQ, NB), 1)
    hot = jnp.zeros((TQ, NB), jnp.float32)
    work = imp
    for _ in range(TOPK):
        mx = jnp.max(work, axis=-1, keepdims=True)
        pick = jnp.min(jnp.where(work == mx, lane, NB), axis=-1, keepdims=True)
        chosen = lane == pick
        hot = jnp.where(chosen, 1.0, hot)
        work = jnp.where(chosen, NEG, work)

    selmask = _dot(hot, exp_ref[...]) > 0.5

    start = pl.multiple_of(jnp.maximum(q0 - WIN, 0), TQ)
    brows = q0 + jax.lax.broadcasted_iota(jnp.int32, (TQ, BAND), 0)
    bcols = start + jax.lax.broadcasted_iota(jnp.int32, (TQ, BAND), 1)
    bwmask = (bcols <= brows) & (brows - bcols <= WIN)

    g0 = g_ref[:, 0:1]
    g1 = g_ref[:, 1:2]
    g2 = g_ref[:, 2:3]
    chs = []
    for h in range(H):
        s = _dot_t(q_ref[h], k_ref[h])
        m = jnp.max(s, axis=-1, keepdims=True)
        p = jnp.exp(s - m).astype(bf)
        zero = jnp.zeros((), bf)
        psel = jnp.where(selmask, p, zero)
        kband = k_ref[h, pl.ds(start, BAND), :]
        vband = v_ref[h, pl.ds(start, BAND), :]
        sband = _dot_t(q_ref[h], kband)
        pwin = jnp.where(bwmask, jnp.exp(sband - m).astype(bf), zero)
        rsel = jnp.dot(psel, v_ref[h], preferred_element_type=f32)
        rwin = jnp.dot(pwin, vband, preferred_element_type=f32)
        osel = rsel[:, :HD] / rsel[:, HD:HD + 1]
        owin = rwin[:, :HD] / rwin[:, HD:HD + 1]
        chs.append((g0 * comp[h] + g1 * osel + g2 * owin).astype(bf))
    cat = jnp.concatenate(chs, axis=-1)
    o_ref[...] = jnp.dot(cat, wo_ref[...],
                         preferred_element_type=f32) + bo_ref[0:1, :]


@jax.jit
def _nsa_forward(x, Wq, Wk, Wv, Wo, bo, Wg, bg, wkc, wvc, wpe):
    f32 = jnp.float32
    bf = jnp.bfloat16
    x2 = x.reshape(S, D).astype(bf)
    wqf = Wq.reshape(D, D).astype(bf)
    wkf = Wk.reshape(D, D).astype(bf)
    wvf = Wv.reshape(D, D).astype(bf)
    wo2 = Wo.reshape(D, D).astype(bf)
    eye = jnp.eye(TQ // STRIDE, dtype=f32)
    pk = jnp.kron(eye, wkc.reshape(1, CB)).astype(bf)
    pv = jnp.kron(eye, wvc.reshape(1, CB)).astype(bf)
    wpe_h = wpe.reshape(CB, H, HD).transpose(1, 0, 2)
    wpe_t = jnp.tile(wpe_h, (1, TQ // CB, 1))
    wg_pad = jnp.zeros((D, 128), bf).at[:, :3].set(Wg.astype(bf))
    bg_pad = jnp.full((1, 128), -1e9, f32).at[0, :3].set(bg)
    bo_t = jnp.broadcast_to(bo.reshape(1, D), (8, D))
    bb = np.arange(NB)[:, None]
    tt = np.arange(S)[None, :]
    expand = jnp.asarray(((tt // STRIDE == bb) & (tt % STRIDE < SB)), dtype=bf)

    full = lambda shape: pl.BlockSpec(shape, lambda i: (0,) * len(shape))
    qh, kh, vh, ckh, cvh, g = pl.pallas_call(
        _proj_kernel,
        grid=(NT,),
        in_specs=[
            pl.BlockSpec((TQ, D), lambda i: (i, 0)),
            full((D, D)), full((D, D)), full((D, D)),
            full((CB, TQ)), full((CB, TQ)),
            full((H, TQ, HD)),
            full((D, 128)), full((1, 128)),
        ],
        out_specs=[
            pl.BlockSpec((H, TQ, HD), lambda i: (0, i, 0)),
            pl.BlockSpec((H, TQ, HD), lambda i: (0, i, 0)),
            pl.BlockSpec((H, TQ, 2 * HD), lambda i: (0, i, 0)),
            pl.BlockSpec((H, TQ // STRIDE, HD), lambda i: (0, i, 0)),
            pl.BlockSpec((H, TQ // STRIDE, 2 * HD), lambda i: (0, i, 0)),
            pl.BlockSpec((TQ, 128), lambda i: (i, 0)),
        ],
        out_shape=[
            jax.ShapeDtypeStruct((H, S, HD), bf),
            jax.ShapeDtypeStruct((H, S, HD), bf),
            jax.ShapeDtypeStruct((H, S, 2 * HD), bf),
            jax.ShapeDtypeStruct((H, NB, HD), bf),
            jax.ShapeDtypeStruct((H, NB, 2 * HD), bf),
            jax.ShapeDtypeStruct((S, 128), f32),
        ],
    )(x2, wqf, wkf, wvf, pk, pv, wpe_t, wg_pad, bg_pad)

    out = pl.pallas_call(
        _attn_kernel,
        grid=(NT,),
        in_specs=[
            pl.BlockSpec((H, TQ, HD), lambda i: (0, i, 0)),
            full((H, S, HD)), full((H, S, 2 * HD)),
            full((H, NB, HD)), full((H, NB, 2 * HD)),
            pl.BlockSpec((TQ, 128), lambda i: (i, 0)),
            full((NB, S)),
            full((D, D)), full((8, D)),
        ],
        out_specs=pl.BlockSpec((TQ, D), lambda i: (i, 0)),
        out_shape=jax.ShapeDtypeStruct((S, D), f32),
    )(qh, kh, vh, ckh, cvh, g, expand, wo2, bo_t)
    return out.reshape(1, S, D)


def kernel(x, Wq, Wk, Wv, Wo, bo, Wg, bg, w_k_compress, w_v_compress,
           w_pe_compress):
    return _nsa_forward(x, Wq, Wk, Wv, Wo, bo, Wg, bg,
                        w_k_compress, w_v_compress, w_pe_compress)


# speedup vs baseline: 27.6528x; 1.0118x over previous
import jax
import jax.numpy as jnp
import numpy as np
from jax.experimental import pallas as pl

S = 2048
D = 768
H = 12
HD = 64
CB = 16
STRIDE = 16
SB = 8
TOPK = 4
WIN = 256
NB = (S - CB) // STRIDE + 1
TQ = 256
NT = S // TQ
SCALE = 1.0 / np.sqrt(HD)
NEG = -1e30
BAND = WIN + TQ


def _softmax(x):
    m = jnp.max(x, axis=-1, keepdims=True)
    e = jnp.exp(x - m)
    return e / jnp.sum(e, axis=-1, keepdims=True)


def _dot(a, b):
    return jnp.dot(a.astype(jnp.bfloat16), b.astype(jnp.bfloat16),
                   preferred_element_type=jnp.float32)


def _dot_t(a, b):
    return jax.lax.dot_general(
        a.astype(jnp.bfloat16), b.astype(jnp.bfloat16),
        (((1,), (1,)), ((), ())), preferred_element_type=jnp.float32)


def _proj_kernel(x_ref, wq_ref, wk_ref, wv_ref, pk_ref, pv_ref,
                 wpe_ref, wg_ref, bg_ref,
                 q_out, k_out, v_out, ck_out, cv_out, g_out):
    xt = x_ref[...]
    bf = jnp.bfloat16
    one_s = (jax.lax.broadcasted_iota(jnp.int32, (TQ, HD), 1) == 0).astype(bf)
    one_c = (jax.lax.broadcasted_iota(
        jnp.int32, (TQ // STRIDE, HD), 1) == 0).astype(bf)
    qa = _dot(xt, wq_ref[...])
    ka = _dot(xt, wk_ref[...])
    va = _dot(xt, wv_ref[...])
    cka = _dot(pk_ref[...], ka + wpe_ref[...])
    cva = _dot(pv_ref[...], va + wpe_ref[...])
    for h in range(H):
        qh = qa[:, h * HD:(h + 1) * HD]
        vh = va[:, h * HD:(h + 1) * HD]
        q_out[h] = (qh * SCALE).astype(bf)
        k_out[h] = ka[:, h * HD:(h + 1) * HD].astype(bf)
        v_out[h] = jnp.concatenate([vh.astype(bf), one_s], axis=-1)
        ck_out[h] = cka[:, h * HD:(h + 1) * HD].astype(bf)
        cv_out[h] = jnp.concatenate(
            [cva[:, h * HD:(h + 1) * HD].astype(bf), one_c], axis=-1)
    gl = _dot(xt, wg_ref[...]) + bg_ref[0:1, :]
    g_out[...] = _softmax(gl)


def _attn_kernel(q_ref, k_ref, v_ref, ck_ref, cv_ref, g_ref, exp_ref,
                 wo_ref, bo_ref, o_ref):
    bf = jnp.bfloat16
    f32 = jnp.float32
    q0 = pl.program_id(0) * TQ

    imp = jnp.zeros((TQ, NB), jnp.float32)
    comp = []
    for h in range(H):
        s = _dot_t(q_ref[h], ck_ref[h])
        imp = imp + s
        pc = jnp.exp(s - jnp.max(s, axis=-1, keepdims=True)).astype(bf)
        rc = jnp.dot(pc, cv_ref[h], preferred_element_type=f32)
        comp.append(rc[:, :HD] / rc[:, HD:HD + 1])

    lane = jax.lax.broadcasted_iota(jnp.int32, (TQ, NB), 1)
    hot = jnp.zeros((TQ, NB), jnp.float32)
    work = imp
    for _ in range(TOPK):
        mx = jnp.max(work, axis=-1, keepdims=True)
        pick = jnp.min(jnp.where(work == mx, lane, NB), axis=-1, keepdims=True)
        chosen = lane == pick
        hot = jnp.where(chosen, 1.0, hot)
        work = jnp.where(chosen, NEG, work)

    selmask = _dot(hot, exp_ref[...]) > 0.5

    start = pl.multiple_of(jnp.maximum(q0 - WIN, 0), TQ)
    brows = q0 + jax.lax.broadcasted_iota(jnp.int32, (TQ, BAND), 0)
    bcols = start + jax.lax.broadcasted_iota(jnp.int32, (TQ, BAND), 1)
    bwmask = (bcols <= brows) & (brows - bcols <= WIN)

    g0 = g_ref[:, 0:1]
    g1 = g_ref[:, 1:2]
    g2 = g_ref[:, 2:3]
    chs = []
    for h in range(H):
        s = _dot_t(q_ref[h], k_ref[h])
        m = jnp.max(s, axis=-1, keepdims=True)
        p = jnp.exp(s - m).astype(bf)
        zero = jnp.zeros((), bf)
        psel = jnp.where(selmask, p, zero)
        kband = k_ref[h, pl.ds(start, BAND), :]
        vband = v_ref[h, pl.ds(start, BAND), :]
        sband = _dot_t(q_ref[h], kband)
        pwin = jnp.where(bwmask, jnp.exp(sband - m).astype(bf), zero)
        rsel = jnp.dot(psel, v_ref[h], preferred_element_type=f32)
        rwin = jnp.dot(pwin, vband, preferred_element_type=f32)
        osel = rsel[:, :HD] / rsel[:, HD:HD + 1]
        owin = rwin[:, :HD] / rwin[:, HD:HD + 1]
        chs.append((g0 * comp[h] + g1 * osel + g2 * owin).astype(bf))
    cat = jnp.concatenate(chs, axis=-1)
    o_ref[...] = jnp.dot(cat, wo_ref[...],
                         preferred_element_type=f32) + bo_ref[0:1, :]


@jax.jit
def _nsa_forward(x, Wq, Wk, Wv, Wo, bo, Wg, bg, wkc, wvc, wpe):
    f32 = jnp.float32
    bf = jnp.bfloat16
    x2 = x.reshape(S, D).astype(bf)
    wqf = Wq.reshape(D, D).astype(bf)
    wkf = Wk.reshape(D, D).astype(bf)
    wvf = Wv.reshape(D, D).astype(bf)
    wo2 = Wo.reshape(D, D).astype(bf)
    eye = jnp.eye(TQ // STRIDE, dtype=f32)
    pk = jnp.kron(eye, wkc.reshape(1, CB)).astype(bf)
    pv = jnp.kron(eye, wvc.reshape(1, CB)).astype(bf)
    wpe_t = jnp.tile(wpe.reshape(CB, H * HD), (TQ // CB, 1))
    wg_pad = jnp.zeros((D, 128), bf).at[:, :3].set(Wg.astype(bf))
    bg_pad = jnp.full((1, 128), -1e9, f32).at[0, :3].set(bg)
    bo_t = jnp.broadcast_to(bo.reshape(1, D), (8, D))
    bb = np.arange(NB)[:, None]
    tt = np.arange(S)[None, :]
    expand = jnp.asarray(((tt // STRIDE == bb) & (tt % STRIDE < SB)), dtype=bf)

    full = lambda shape: pl.BlockSpec(shape, lambda i: (0,) * len(shape))
    qh, kh, vh, ckh, cvh, g = pl.pallas_call(
        _proj_kernel,
        grid=(NT,),
        in_specs=[
            pl.BlockSpec((TQ, D), lambda i: (i, 0)),
            full((D, D)), full((D, D)), full((D, D)),
            full((CB, TQ)), full((CB, TQ)),
            full((TQ, D)),
            full((D, 128)), full((1, 128)),
        ],
        out_specs=[
            pl.BlockSpec((H, TQ, HD), lambda i: (0, i, 0)),
            pl.BlockSpec((H, TQ, HD), lambda i: (0, i, 0)),
            pl.BlockSpec((H, TQ, 2 * HD), lambda i: (0, i, 0)),
            pl.BlockSpec((H, TQ // STRIDE, HD), lambda i: (0, i, 0)),
            pl.BlockSpec((H, TQ // STRIDE, 2 * HD), lambda i: (0, i, 0)),
            pl.BlockSpec((TQ, 128), lambda i: (i, 0)),
        ],
        out_shape=[
            jax.ShapeDtypeStruct((H, S, HD), bf),
            jax.ShapeDtypeStruct((H, S, HD), bf),
            jax.ShapeDtypeStruct((H, S, 2 * HD), bf),
            jax.ShapeDtypeStruct((H, NB, HD), bf),
            jax.ShapeDtypeStruct((H, NB, 2 * HD), bf),
            jax.ShapeDtypeStruct((S, 128), f32),
        ],
    )(x2, wqf, wkf, wvf, pk, pv, wpe_t, wg_pad, bg_pad)

    out = pl.pallas_call(
        _attn_kernel,
        grid=(NT,),
        in_specs=[
            pl.BlockSpec((H, TQ, HD), lambda i: (0, i, 0)),
            full((H, S, HD)), full((H, S, 2 * HD)),
            full((H, NB, HD)), full((H, NB, 2 * HD)),
            pl.BlockSpec((TQ, 128), lambda i: (i, 0)),
            full((NB, S)),
            full((D, D)), full((8, D)),
        ],
        out_specs=pl.BlockSpec((TQ, D), lambda i: (i, 0)),
        out_shape=jax.ShapeDtypeStruct((S, D), f32),
    )(qh, kh, vh, ckh, cvh, g, expand, wo2, bo_t)
    return out.reshape(1, S, D)


def kernel(x, Wq, Wk, Wv, Wo, bo, Wg, bg, w_k_compress, w_v_compress,
           w_pe_compress):
    return _nsa_forward(x, Wq, Wk, Wv, Wo, bo, Wg, bg,
                        w_k_compress, w_v_compress, w_pe_compress)
